# baseline (speedup 1.0000x reference)
import functools

import jax
import jax.numpy as jnp
import numpy as np
from jax import lax
from jax.experimental import pallas as pl
from jax.experimental.pallas import tpu as pltpu
from jax.experimental.pallas import tpu_sc as plsc

_BIDX = np.array([i % 8 for i in range(64)], dtype=np.int64)
_XC = np.array([32 * i for i in range(64)], dtype=np.int64)
_YC = np.array([16 * i + 8 for i in range(64)], dtype=np.int64)
_FLAT_IDX = jnp.asarray(
    (_BIDX * 2048 * 2048 + _YC * 2048 + _XC).astype(np.int32)
)

_N = 64

_mesh = plsc.VectorSubcoreMesh(core_axis_name="c", subcore_axis_name="s")


@functools.partial(
    pl.kernel,
    out_type=jax.ShapeDtypeStruct((_N,), jnp.float32),
    mesh=_mesh,
    scratch_types=[
        pltpu.VMEM((_N,), jnp.int32),
        pltpu.VMEM((_N,), jnp.float32),
        pltpu.SemaphoreType.DMA,
    ],
)
def _probe_gather(x_hbm, idx_hbm, out_hbm, idx_v, vals_v, sem):
    wid = lax.axis_index("s") * 2 + lax.axis_index("c")

    @pl.when(wid == 0)
    def _():
        pltpu.sync_copy(idx_hbm, idx_v)
        pltpu.async_copy(x_hbm.at[idx_v], vals_v, sem).wait()
        pltpu.sync_copy(vals_v, out_hbm)


def kernel(x):
    return _probe_gather(x.reshape(-1), _FLAT_IDX)


# speedup vs baseline: 6.2228x; 6.2228x over previous
import functools

import jax
import jax.numpy as jnp
from jax import lax
from jax.experimental import pallas as pl
from jax.experimental.pallas import tpu as pltpu
from jax.experimental.pallas import tpu_sc as plsc

_N = 64
_NT = 8
_PPT = _N // _NT

_mesh = plsc.VectorSubcoreMesh(core_axis_name="c", subcore_axis_name="s")


@functools.partial(
    pl.kernel,
    out_type=jax.ShapeDtypeStruct((_N,), jnp.float32),
    mesh=_mesh,
    scratch_types=[
        pltpu.VMEM((_PPT * 8,), jnp.float32),
        pltpu.VMEM((16,), jnp.float32),
        pltpu.SemaphoreType.DMA,
    ],
)
def _probe_gather(x_hbm, out_hbm, rows_v, out_v, sem):
    wid = lax.axis_index("s") * 2 + lax.axis_index("c")

    @pl.when(wid < _NT)
    def _():
        copies = []
        for j in range(_PPT):
            y = 16 * _PPT * wid + 16 * j + 8
            c = 32 * _PPT * wid + 32 * j
            copies.append(
                pltpu.async_copy(
                    x_hbm.at[j, y, pl.ds(c, 8)], rows_v.at[pl.ds(8 * j, 8)], sem
                )
            )
        for cp in copies:
            cp.wait()
        lane = lax.iota(jnp.int32, 16)
        vals = jnp.zeros((16,), jnp.float32)
        for j in range(_PPT):
            v = rows_v[pl.ds(16 * (j // 2), 16)]
            vals = jnp.where(lane == j, v[8 * (j % 2)], vals)
        out_v[...] = vals
        pltpu.sync_copy(
            out_v.at[pl.ds(0, _PPT)], out_hbm.at[pl.ds(_PPT * wid, _PPT)]
        )


def kernel(x):
    return _probe_gather(x)
